# baseline (speedup 1.0000x reference)
import functools

import jax
import jax.numpy as jnp
from jax import lax
from jax.experimental import pallas as pl
from jax.experimental.pallas import tpu as pltpu
from jax.experimental.pallas import tpu_sc as plsc

VOCAB = 100000
EMBED = 200
HIDDEN = 64
CLASSES = 100
BATCH = 4096
SEQ = 50

LANES = 16
NCORES = 2
NSUB = 16
NWORKERS = NCORES * NSUB
ROWS_PER_W = BATCH // NWORKERS
NCHUNK = EMBED // LANES
TAIL_OFF = EMBED - LANES


def _pool_body(x_hbm, embed_hbm, out_hbm, idx_v, rows_a, rows_b, out_v,
               sem_a, sem_b):
    cid = lax.axis_index("c")
    sid = lax.axis_index("s")
    wid = sid * NCORES + cid
    base = wid * ROWS_PER_W

    pltpu.sync_copy(x_hbm.at[pl.ds(base, ROWS_PER_W)], idx_v)

    pltpu.async_copy(embed_hbm.at[idx_v.at[0]], rows_a, sem_a)

    inv = jnp.full((LANES,), 1.0 / SEQ, dtype=jnp.float32)

    def accumulate(buf, i):
        accs = [buf[0, pl.ds(k * LANES, LANES)] for k in range(NCHUNK)]
        acc_t = buf[0, pl.ds(TAIL_OFF, LANES)]
        for j in range(1, SEQ):
            accs = [accs[k] + buf[j, pl.ds(k * LANES, LANES)]
                    for k in range(NCHUNK)]
            acc_t = acc_t + buf[j, pl.ds(TAIL_OFF, LANES)]
        for k in range(NCHUNK):
            out_v[i, pl.ds(k * LANES, LANES)] = accs[k] * inv
        out_v[i, pl.ds(TAIL_OFF, LANES)] = acc_t * inv

    def loop_body(j, carry):
        for b, (cur, cur_sem, nxt, nxt_sem) in enumerate(
                ((rows_a, sem_a, rows_b, sem_b),
                 (rows_b, sem_b, rows_a, sem_a))):
            i = j + b
            nxt_row = jnp.minimum(i + 1, ROWS_PER_W - 1)
            pltpu.async_copy(embed_hbm.at[idx_v.at[nxt_row]], nxt, nxt_sem)
            pltpu.make_async_copy(embed_hbm.at[idx_v.at[i]], cur,
                                  cur_sem).wait()
            accumulate(cur, i)
        return carry

    lax.fori_loop(0, ROWS_PER_W // 2, lambda t, c: loop_body(t * 2, c), 0,
                  unroll=False)

    pltpu.make_async_copy(embed_hbm.at[idx_v.at[0]], rows_a, sem_a).wait()

    pltpu.sync_copy(out_v, out_hbm.at[pl.ds(base, ROWS_PER_W)])


@functools.partial(
    pl.kernel,
    out_type=jax.ShapeDtypeStruct((BATCH, EMBED), jnp.float32),
    mesh=plsc.VectorSubcoreMesh(core_axis_name="c", subcore_axis_name="s"),
    compiler_params=pltpu.CompilerParams(use_tc_tiling_on_sc=False),
    scratch_types=[
        pltpu.VMEM((ROWS_PER_W, SEQ), jnp.int32),
        pltpu.VMEM((SEQ, EMBED), jnp.float32),
        pltpu.VMEM((SEQ, EMBED), jnp.float32),
        pltpu.VMEM((ROWS_PER_W, EMBED), jnp.float32),
        pltpu.SemaphoreType.DMA,
        pltpu.SemaphoreType.DMA,
    ],
)
def _pool(x_hbm, embed_hbm, out_hbm, idx_v, rows_a, rows_b, out_v,
          sem_a, sem_b):
    _pool_body(x_hbm, embed_hbm, out_hbm, idx_v, rows_a, rows_b, out_v,
               sem_a, sem_b)


MLP_BLK = 1024


def _mlp_body(p_ref, w1_ref, b1_ref, w2_ref, b2_ref, o_ref):
    h = jnp.dot(p_ref[...], w1_ref[...],
                preferred_element_type=jnp.float32) + b1_ref[...]
    h = jnp.maximum(h, 0.0)
    z = jnp.dot(h, w2_ref[...],
                preferred_element_type=jnp.float32) + b2_ref[...]
    m = jnp.max(z, axis=1, keepdims=True)
    lse = jnp.log(jnp.sum(jnp.exp(z - m), axis=1, keepdims=True)) + m
    o_ref[...] = z - lse


def _mlp(pooled, W1, b1, W2, b2):
    return pl.pallas_call(
        _mlp_body,
        grid=(BATCH // MLP_BLK,),
        in_specs=[
            pl.BlockSpec((MLP_BLK, EMBED), lambda i: (i, 0)),
            pl.BlockSpec((EMBED, HIDDEN), lambda i: (0, 0)),
            pl.BlockSpec((1, HIDDEN), lambda i: (0, 0)),
            pl.BlockSpec((HIDDEN, CLASSES), lambda i: (0, 0)),
            pl.BlockSpec((1, CLASSES), lambda i: (0, 0)),
        ],
        out_specs=pl.BlockSpec((MLP_BLK, CLASSES), lambda i: (i, 0)),
        out_shape=jax.ShapeDtypeStruct((BATCH, CLASSES), jnp.float32),
    )(pooled, W1, b1.reshape(1, HIDDEN), W2, b2.reshape(1, CLASSES))


def kernel(x, embed, W1, b1, W2, b2):
    xi = x.astype(jnp.int32)
    one = lax.optimization_barrier(jnp.float32(1.0))
    embed_lin = embed * one
    pooled = _pool(xi, embed_lin)
    return _mlp(pooled, W1, b1, W2, b2)


# speedup vs baseline: 1.1479x; 1.1479x over previous
import functools

import jax
import jax.numpy as jnp
from jax import lax
from jax.experimental import pallas as pl
from jax.experimental.pallas import tpu as pltpu
from jax.experimental.pallas import tpu_sc as plsc

VOCAB = 100000
EMBED = 200
EMBED_P = 256
HIDDEN = 64
CLASSES = 100
BATCH = 4096
SEQ = 50

LANES = 16
NCORES = 2
NSUB = 16
NWORKERS = NCORES * NSUB
ROWS_PER_W = BATCH // NWORKERS
NCHUNK = EMBED // LANES
TAIL_OFF = EMBED - LANES


def _pool_body(x_hbm, embed_hbm, out_hbm, idx_v, rows_a, rows_b, out_v,
               sem_a, sem_b):
    cid = lax.axis_index("c")
    sid = lax.axis_index("s")
    wid = sid * NCORES + cid
    base = wid * ROWS_PER_W

    pltpu.sync_copy(x_hbm.at[pl.ds(base, ROWS_PER_W)], idx_v)

    pltpu.async_copy(embed_hbm.at[idx_v.at[0]], rows_a, sem_a)

    inv = jnp.full((LANES,), 1.0 / SEQ, dtype=jnp.float32)

    def accumulate(buf, i):
        accs = [buf[0, pl.ds(k * LANES, LANES)] for k in range(NCHUNK)]
        acc_t = buf[0, pl.ds(TAIL_OFF, LANES)]
        for j in range(1, SEQ):
            accs = [accs[k] + buf[j, pl.ds(k * LANES, LANES)]
                    for k in range(NCHUNK)]
            acc_t = acc_t + buf[j, pl.ds(TAIL_OFF, LANES)]
        for k in range(NCHUNK):
            out_v[i, pl.ds(k * LANES, LANES)] = accs[k] * inv
        out_v[i, pl.ds(TAIL_OFF, LANES)] = acc_t * inv

    def loop_body(j, carry):
        for b, (cur, cur_sem, nxt, nxt_sem) in enumerate(
                ((rows_a, sem_a, rows_b, sem_b),
                 (rows_b, sem_b, rows_a, sem_a))):
            i = j + b
            nxt_row = jnp.minimum(i + 1, ROWS_PER_W - 1)
            pltpu.async_copy(embed_hbm.at[idx_v.at[nxt_row]], nxt, nxt_sem)
            pltpu.make_async_copy(embed_hbm.at[idx_v.at[i]], cur,
                                  cur_sem).wait()
            accumulate(cur, i)
        return carry

    lax.fori_loop(0, ROWS_PER_W // 2, lambda t, c: loop_body(t * 2, c), 0,
                  unroll=False)

    pltpu.make_async_copy(embed_hbm.at[idx_v.at[0]], rows_a, sem_a).wait()

    pltpu.sync_copy(out_v, out_hbm.at[pl.ds(base, ROWS_PER_W)])


@functools.partial(
    pl.kernel,
    out_type=jax.ShapeDtypeStruct((BATCH, EMBED), jnp.float32),
    mesh=plsc.VectorSubcoreMesh(core_axis_name="c", subcore_axis_name="s"),
    compiler_params=pltpu.CompilerParams(use_tc_tiling_on_sc=True),
    scratch_types=[
        pltpu.VMEM((ROWS_PER_W, SEQ), jnp.int32),
        pltpu.VMEM((SEQ, EMBED_P), jnp.float32),
        pltpu.VMEM((SEQ, EMBED_P), jnp.float32),
        pltpu.VMEM((ROWS_PER_W, EMBED), jnp.float32),
        pltpu.SemaphoreType.DMA,
        pltpu.SemaphoreType.DMA,
    ],
)
def _pool(x_hbm, embed_hbm, out_hbm, idx_v, rows_a, rows_b, out_v,
          sem_a, sem_b):
    _pool_body(x_hbm, embed_hbm, out_hbm, idx_v, rows_a, rows_b, out_v,
               sem_a, sem_b)


MLP_BLK = 1024


def _mlp_body(p_ref, w1_ref, b1_ref, w2_ref, b2_ref, o_ref):
    h = jnp.dot(p_ref[...], w1_ref[...],
                preferred_element_type=jnp.float32) + b1_ref[...]
    h = jnp.maximum(h, 0.0)
    z = jnp.dot(h, w2_ref[...],
                preferred_element_type=jnp.float32) + b2_ref[...]
    m = jnp.max(z, axis=1, keepdims=True)
    lse = jnp.log(jnp.sum(jnp.exp(z - m), axis=1, keepdims=True)) + m
    o_ref[...] = z - lse


def _mlp(pooled, W1, b1, W2, b2):
    return pl.pallas_call(
        _mlp_body,
        grid=(BATCH // MLP_BLK,),
        in_specs=[
            pl.BlockSpec((MLP_BLK, EMBED), lambda i: (i, 0)),
            pl.BlockSpec((EMBED, HIDDEN), lambda i: (0, 0)),
            pl.BlockSpec((1, HIDDEN), lambda i: (0, 0)),
            pl.BlockSpec((HIDDEN, CLASSES), lambda i: (0, 0)),
            pl.BlockSpec((1, CLASSES), lambda i: (0, 0)),
        ],
        out_specs=pl.BlockSpec((MLP_BLK, CLASSES), lambda i: (i, 0)),
        out_shape=jax.ShapeDtypeStruct((BATCH, CLASSES), jnp.float32),
    )(pooled, W1, b1.reshape(1, HIDDEN), W2, b2.reshape(1, CLASSES))


def kernel(x, embed, W1, b1, W2, b2):
    xi = x.astype(jnp.int32)
    embed_p = jnp.pad(embed, ((0, 0), (0, EMBED_P - EMBED)))
    pooled = _pool(xi, embed_p)
    return _mlp(pooled, W1, b1, W2, b2)
